# baseline (speedup 1.0000x reference)
import functools

import jax
import jax.numpy as jnp
from jax import lax
from jax.experimental import pallas as pl
from jax.experimental.pallas import tpu as pltpu
from jax.experimental.pallas import tpu_sc as plsc

_N = 10000
_NP = 10112
_E = 320000
_D = 128
_K = 40
_NC, _NS = 2, 16
_NW = _NC * _NS
_CH = 250
_SECS = (128, 122)
_RPT = _NP // _NS


@functools.partial(
    pl.kernel,
    mesh=plsc.VectorSubcoreMesh(core_axis_name="c", subcore_axis_name="s"),
    out_type=jax.ShapeDtypeStruct((_NC, _NP, _D), jnp.float32),
    scratch_types=[
        pltpu.VMEM((_SECS[0], _K), jnp.int32),
        pltpu.VMEM((_SECS[0], _K), jnp.int32),
        pltpu.VMEM((2, _K, _D), jnp.float32),
        pltpu.VMEM_SHARED((_NP, _D), jnp.float32),
        pltpu.SemaphoreType.DMA,
    ],
)
def _sc_segment_sum(src_hbm, dst_hbm, table_hbm, zeros_hbm, out_hbm,
                    sidx, didx, rows, acc, gsem):
    c = lax.axis_index("c")
    s = lax.axis_index("s")
    wid = s * _NC + c
    pltpu.sync_copy(zeros_hbm.at[pl.ds(s * _RPT, _RPT)],
                    acc.at[pl.ds(s * _RPT, _RPT)])
    plsc.subcore_barrier()

    def step(j, b, length):
        pltpu.make_async_copy(table_hbm.at[sidx.at[j]], rows.at[b], gsem).wait()

        @pl.when(j + 1 < length)
        def _():
            pltpu.async_copy(table_hbm.at[sidx.at[j + 1]], rows.at[1 - b], gsem)

        pltpu.sync_copy(rows.at[b], acc.at[didx.at[j]], add=True)

    off = 0
    for length in _SECS:
        pltpu.sync_copy(src_hbm.at[wid, pl.ds(off, length)],
                        sidx.at[pl.ds(0, length)])
        pltpu.sync_copy(dst_hbm.at[wid, pl.ds(off, length)],
                        didx.at[pl.ds(0, length)])
        pltpu.async_copy(table_hbm.at[sidx.at[0]], rows.at[0], gsem)

        def outer(i, carry, length=length):
            step(2 * i, 0, length)
            step(2 * i + 1, 1, length)
            return carry

        lax.fori_loop(0, length // 2, outer, 0)
        if length % 2:
            step(length - 1, 0, length)
        off += length

    plsc.subcore_barrier()
    pltpu.sync_copy(acc.at[pl.ds(s * _RPT, _RPT)],
                    out_hbm.at[c, pl.ds(s * _RPT, _RPT)])


def _combine_body(p_ref, o_ref):
    o_ref[...] = p_ref[0] + p_ref[1]


_ROWS_PER_BLK = 1000


def _combine(partials):
    return pl.pallas_call(
        _combine_body,
        out_shape=jax.ShapeDtypeStruct((_N, _D), jnp.float32),
        grid=(_N // _ROWS_PER_BLK,),
        in_specs=[pl.BlockSpec((_NC, _ROWS_PER_BLK, _D), lambda i: (0, i, 0))],
        out_specs=pl.BlockSpec((_ROWS_PER_BLK, _D), lambda i: (i, 0)),
    )(partials)


def kernel(source, target, edge_index, W_emb, b_emb, W_loc, b_loc, W_nb, b_nb):
    src3d = edge_index[0].reshape(_NW, _CH, _K)
    dst3d = edge_index[1].reshape(_NW, _CH, _K)
    zeros = jnp.zeros((_NP, _D), jnp.float32)
    partials = _sc_segment_sum(src3d, dst3d, source, zeros)
    return _combine(partials)


# speedup vs baseline: 1.5025x; 1.5025x over previous
import functools

import jax
import jax.numpy as jnp
from jax import lax
from jax.experimental import pallas as pl
from jax.experimental.pallas import tpu as pltpu
from jax.experimental.pallas import tpu_sc as plsc

_N = 10000
_NP = 10112
_E = 320000
_D = 128
_K = 100
_NC, _NS = 2, 16
_NW = _NC * _NS
_CH = 100
_SECS = (56, 44)
_RPT = _NP // _NS


@functools.partial(
    pl.kernel,
    mesh=plsc.VectorSubcoreMesh(core_axis_name="c", subcore_axis_name="s"),
    out_type=jax.ShapeDtypeStruct((_NC, _NP, _D), jnp.float32),
    scratch_types=[
        pltpu.VMEM((_SECS[0], _K), jnp.int32),
        pltpu.VMEM((_SECS[0], _K), jnp.int32),
        pltpu.VMEM((2, _K, _D), jnp.float32),
        pltpu.VMEM_SHARED((_NP, _D), jnp.float32),
        pltpu.SemaphoreType.DMA,
    ],
)
def _sc_segment_sum(src_hbm, dst_hbm, table_hbm, zeros_hbm, out_hbm,
                    sidx, didx, rows, acc, gsem):
    c = lax.axis_index("c")
    s = lax.axis_index("s")
    wid = s * _NC + c
    pltpu.sync_copy(zeros_hbm.at[pl.ds(s * _RPT, _RPT)],
                    acc.at[pl.ds(s * _RPT, _RPT)])
    plsc.subcore_barrier()

    def step(j, b, length):
        pltpu.make_async_copy(table_hbm.at[sidx.at[j]], rows.at[b], gsem).wait()

        @pl.when(j + 1 < length)
        def _():
            pltpu.async_copy(table_hbm.at[sidx.at[j + 1]], rows.at[1 - b], gsem)

        pltpu.sync_copy(rows.at[b], acc.at[didx.at[j]], add=True)

    off = 0
    for length in _SECS:
        pltpu.sync_copy(src_hbm.at[wid, pl.ds(off, length)],
                        sidx.at[pl.ds(0, length)])
        pltpu.sync_copy(dst_hbm.at[wid, pl.ds(off, length)],
                        didx.at[pl.ds(0, length)])
        pltpu.async_copy(table_hbm.at[sidx.at[0]], rows.at[0], gsem)

        def outer(i, carry, length=length):
            step(2 * i, 0, length)
            step(2 * i + 1, 1, length)
            return carry

        lax.fori_loop(0, length // 2, outer, 0)
        if length % 2:
            step(length - 1, 0, length)
        off += length

    plsc.subcore_barrier()
    pltpu.sync_copy(acc.at[pl.ds(s * _RPT, _RPT)],
                    out_hbm.at[c, pl.ds(s * _RPT, _RPT)])


def _combine_body(p_ref, o_ref):
    o_ref[...] = p_ref[0] + p_ref[1]


_ROWS_PER_BLK = 1000


def _combine(partials):
    return pl.pallas_call(
        _combine_body,
        out_shape=jax.ShapeDtypeStruct((_N, _D), jnp.float32),
        grid=(_N // _ROWS_PER_BLK,),
        in_specs=[pl.BlockSpec((_NC, _ROWS_PER_BLK, _D), lambda i: (0, i, 0))],
        out_specs=pl.BlockSpec((_ROWS_PER_BLK, _D), lambda i: (i, 0)),
    )(partials)


def kernel(source, target, edge_index, W_emb, b_emb, W_loc, b_loc, W_nb, b_nb):
    src3d = edge_index[0].reshape(_NW, _CH, _K)
    dst3d = edge_index[1].reshape(_NW, _CH, _K)
    zeros = jnp.zeros((_NP, _D), jnp.float32)
    partials = _sc_segment_sum(src3d, dst3d, source, zeros)
    return _combine(partials)


# speedup vs baseline: 2.0298x; 1.3509x over previous
import functools

import jax
import jax.numpy as jnp
from jax import lax
from jax.experimental import pallas as pl
from jax.experimental.pallas import tpu as pltpu
from jax.experimental.pallas import tpu_sc as plsc

_N = 10000
_NP = 10112
_E = 320000
_D = 128
_K = 100
_NC, _NS = 2, 16
_NW = _NC * _NS
_CH = 100
_SECS = (32, 32, 36)
_RPT = _NP // _NS
_NB = 3
_ZR = 96


@functools.partial(
    pl.kernel,
    mesh=plsc.VectorSubcoreMesh(core_axis_name="c", subcore_axis_name="s"),
    out_type=jax.ShapeDtypeStruct((_NC, _NP, _D), jnp.float32),
    scratch_types=[
        pltpu.VMEM((max(_SECS), _K), jnp.int32),
        pltpu.VMEM((max(_SECS), _K), jnp.int32),
        pltpu.VMEM((_NB, _K, _D), jnp.float32),
        pltpu.VMEM_SHARED((_NP, _D), jnp.float32),
        pltpu.SemaphoreType.DMA,
    ],
)
def _sc_segment_sum(src_hbm, dst_hbm, table_hbm, out_hbm,
                    sidx, didx, rows, acc, gsem):
    c = lax.axis_index("c")
    s = lax.axis_index("s")
    wid = s * _NC + c

    zbuf = rows.at[0]
    zeros16 = jnp.zeros((16,), jnp.float32)

    def zrow(r, carry):
        for v in range(_D // 16):
            zbuf[r, pl.ds(v * 16, 16)] = zeros16
        return carry

    lax.fori_loop(0, _K, zrow, 0)
    base = s * _RPT
    for k in range(_RPT // _ZR):
        pltpu.sync_copy(zbuf.at[pl.ds(0, _ZR)],
                        acc.at[pl.ds(base + k * _ZR, _ZR)])
    _tail = _RPT - (_RPT // _ZR) * _ZR
    pltpu.sync_copy(zbuf.at[pl.ds(0, _tail)],
                    acc.at[pl.ds(base + (_RPT // _ZR) * _ZR, _tail)])
    plsc.subcore_barrier()

    def step(j, b, length):
        pltpu.make_async_copy(table_hbm.at[sidx.at[j]], rows.at[b], gsem).wait()

        @pl.when(j + _NB - 1 < length)
        def _():
            pltpu.async_copy(table_hbm.at[sidx.at[j + _NB - 1]],
                             rows.at[(j + _NB - 1) % _NB], gsem)

        pltpu.sync_copy(rows.at[b], acc.at[didx.at[j]], add=True)

    off = 0
    for length in _SECS:
        pltpu.sync_copy(src_hbm.at[wid, pl.ds(off, length)],
                        sidx.at[pl.ds(0, length)])
        pltpu.sync_copy(dst_hbm.at[wid, pl.ds(off, length)],
                        didx.at[pl.ds(0, length)])
        pltpu.async_copy(table_hbm.at[sidx.at[0]], rows.at[0], gsem)
        pltpu.async_copy(table_hbm.at[sidx.at[1]], rows.at[1], gsem)

        def outer(i, carry, length=length):
            for r in range(_NB):
                step(_NB * i + r, r, length)
            return carry

        lax.fori_loop(0, length // _NB, outer, 0)
        for r in range(length % _NB):
            j = (length // _NB) * _NB + r
            step(j, j % _NB, length)
        off += length

    plsc.subcore_barrier()
    pltpu.sync_copy(acc.at[pl.ds(s * _RPT, _RPT)],
                    out_hbm.at[c, pl.ds(s * _RPT, _RPT)])


def _combine_body(p_ref, o_ref):
    o_ref[...] = p_ref[0] + p_ref[1]


_ROWS_PER_BLK = 1000


def _combine(partials):
    return pl.pallas_call(
        _combine_body,
        out_shape=jax.ShapeDtypeStruct((_N, _D), jnp.float32),
        grid=(_N // _ROWS_PER_BLK,),
        in_specs=[pl.BlockSpec((_NC, _ROWS_PER_BLK, _D), lambda i: (0, i, 0))],
        out_specs=pl.BlockSpec((_ROWS_PER_BLK, _D), lambda i: (i, 0)),
    )(partials)


def kernel(source, target, edge_index, W_emb, b_emb, W_loc, b_loc, W_nb, b_nb):
    src3d = edge_index[0].reshape(_NW, _CH, _K)
    dst3d = edge_index[1].reshape(_NW, _CH, _K)
    partials = _sc_segment_sum(src3d, dst3d, source)
    return _combine(partials)


# speedup vs baseline: 2.2073x; 1.0874x over previous
import functools

import jax
import jax.numpy as jnp
from jax import lax
from jax.experimental import pallas as pl
from jax.experimental.pallas import tpu as pltpu
from jax.experimental.pallas import tpu_sc as plsc

_N = 10000
_NP = 10112
_E = 320000
_D = 128
_K = 100
_NC, _NS = 2, 16
_NW = _NC * _NS
_CH = 100
_SECS = (32, 32, 36)
_RPT = _NP // _NS
_NB = 3
_ZR = 96


@functools.partial(
    pl.kernel,
    mesh=plsc.VectorSubcoreMesh(core_axis_name="c", subcore_axis_name="s"),
    out_type=jax.ShapeDtypeStruct((_NC, _NP, _D), jnp.float32),
    scratch_types=[
        pltpu.VMEM((max(_SECS), _K), jnp.int32),
        pltpu.VMEM((max(_SECS), _K), jnp.int32),
        pltpu.VMEM((_NB, _K, _D), jnp.float32),
        pltpu.VMEM_SHARED((_NP, _D), jnp.float32),
        pltpu.SemaphoreType.DMA,
    ],
)
def _sc_segment_sum(idx_hbm, table_hbm, out_hbm,
                    sidx, didx, rows, acc, gsem):
    c = lax.axis_index("c")
    s = lax.axis_index("s")
    wid = s * _NC + c

    zbuf = rows.at[0]
    zeros16 = jnp.zeros((16,), jnp.float32)

    def zrow(r, carry):
        for v in range(_D // 16):
            zbuf[r, pl.ds(v * 16, 16)] = zeros16
        return carry

    lax.fori_loop(0, _K, zrow, 0)
    base = s * _RPT
    for k in range(_RPT // _ZR):
        pltpu.sync_copy(zbuf.at[pl.ds(0, _ZR)],
                        acc.at[pl.ds(base + k * _ZR, _ZR)])
    _tail = _RPT - (_RPT // _ZR) * _ZR
    pltpu.sync_copy(zbuf.at[pl.ds(0, _tail)],
                    acc.at[pl.ds(base + (_RPT // _ZR) * _ZR, _tail)])
    plsc.subcore_barrier()

    def step(j, b, length):
        pltpu.make_async_copy(table_hbm.at[sidx.at[j]], rows.at[b], gsem).wait()

        @pl.when(j + _NB - 1 < length)
        def _():
            pltpu.async_copy(table_hbm.at[sidx.at[j + _NB - 1]],
                             rows.at[(j + _NB - 1) % _NB], gsem)

        pltpu.sync_copy(rows.at[b], acc.at[didx.at[j]], add=True)

    off = 0
    for length in _SECS:
        pltpu.sync_copy(idx_hbm.at[0, wid, pl.ds(off, length)],
                        sidx.at[pl.ds(0, length)])
        pltpu.sync_copy(idx_hbm.at[1, wid, pl.ds(off, length)],
                        didx.at[pl.ds(0, length)])
        pltpu.async_copy(table_hbm.at[sidx.at[0]], rows.at[0], gsem)
        pltpu.async_copy(table_hbm.at[sidx.at[1]], rows.at[1], gsem)

        def outer(i, carry, length=length):
            for r in range(_NB):
                step(_NB * i + r, r, length)
            return carry

        lax.fori_loop(0, length // _NB, outer, 0)
        for r in range(length % _NB):
            j = (length // _NB) * _NB + r
            step(j, j % _NB, length)
        off += length

    plsc.subcore_barrier()
    pltpu.sync_copy(acc.at[pl.ds(s * _RPT, _RPT)],
                    out_hbm.at[c, pl.ds(s * _RPT, _RPT)])


def _combine_body(p_ref, o_ref):
    o_ref[...] = p_ref[0] + p_ref[1]


_ROWS_PER_BLK = 1000


def _combine(partials):
    return pl.pallas_call(
        _combine_body,
        out_shape=jax.ShapeDtypeStruct((_N, _D), jnp.float32),
        grid=(_N // _ROWS_PER_BLK,),
        in_specs=[pl.BlockSpec((_NC, _ROWS_PER_BLK, _D), lambda i: (0, i, 0))],
        out_specs=pl.BlockSpec((_ROWS_PER_BLK, _D), lambda i: (i, 0)),
    )(partials)


def kernel(source, target, edge_index, W_emb, b_emb, W_loc, b_loc, W_nb, b_nb):
    idx4d = edge_index.reshape(2, _NW, _CH, _K)
    partials = _sc_segment_sum(idx4d, source)
    return _combine(partials)
